# baseline (speedup 1.0000x reference)
import functools

import jax
import jax.numpy as jnp
from jax.experimental import pallas as pl

_PBLK = 98


def _pos_kernel(row_ref, col_ref, out_ref, *, h, w, d):
    blk = out_ref.shape[0]
    b = out_ref.shape[1]
    p = _PBLK * pl.program_id(0) + jax.lax.broadcasted_iota(
        jnp.int32, (blk, max(h, w)), 0)
    q = jax.lax.broadcasted_iota(jnp.int32, (blk, max(h, w)), 1)
    sel_col = (p % w == q).astype(jnp.float32)[:, :w]
    sel_row = (p // w == q).astype(jnp.float32)[:, :h]
    top = jax.lax.dot_general(
        sel_col, col_ref[:w, :], (((1,), (0,)), ((), ())),
        preferred_element_type=jnp.float32,
        precision=jax.lax.Precision.HIGHEST)
    bottom = jax.lax.dot_general(
        sel_row, row_ref[:h, :], (((1,), (0,)), ((), ())),
        preferred_element_type=jnp.float32,
        precision=jax.lax.Precision.HIGHEST)
    vec = jnp.concatenate([top, bottom], axis=1)
    out_ref[...] = jnp.broadcast_to(vec[:, None, :], (blk, b, 2 * d))


def kernel(x, row_embed, col_embed):
    b = x.shape[0]
    h, w = x.shape[2], x.shape[3]
    n, d = row_embed.shape
    hw = h * w
    body = functools.partial(_pos_kernel, h=h, w=w, d=d)
    out = pl.pallas_call(
        body,
        grid=(hw // _PBLK,),
        in_specs=[
            pl.BlockSpec((n, d), lambda i: (0, 0)),
            pl.BlockSpec((n, d), lambda i: (0, 0)),
        ],
        out_specs=pl.BlockSpec((_PBLK, b, 2 * d), lambda i: (i, 0, 0)),
        out_shape=jax.ShapeDtypeStruct((hw, b, 2 * d), jnp.float32),
    )(row_embed, col_embed)
    return jnp.transpose(out.reshape(h, w, b, 2 * d), (2, 3, 0, 1))


# speedup vs baseline: 1.0575x; 1.0575x over previous
import functools

import jax
import jax.numpy as jnp
from jax.experimental import pallas as pl
from jax.experimental.pallas import tpu as pltpu

_CHUNK = 28
_NCHUNK = 7


def _pos_kernel(row_ref, col_ref, out_ref, stage, sems, *, h, w, d):
    hw = h * w
    b = out_ref.shape[1]
    for c in range(_NCHUNK):
        base = c * _CHUNK
        p = base + jax.lax.broadcasted_iota(
            jnp.int32, (_CHUNK, max(h, w)), 0)
        q = jax.lax.broadcasted_iota(jnp.int32, (_CHUNK, max(h, w)), 1)
        sel_col = (p % w == q).astype(jnp.float32)[:, :w]
        sel_row = (p // w == q).astype(jnp.float32)[:, :h]
        top = jax.lax.dot_general(
            sel_col, col_ref[:w, :], (((1,), (0,)), ((), ())),
            preferred_element_type=jnp.float32,
            precision=jax.lax.Precision.HIGHEST)
        bottom = jax.lax.dot_general(
            sel_row, row_ref[:h, :], (((1,), (0,)), ((), ())),
            preferred_element_type=jnp.float32,
            precision=jax.lax.Precision.HIGHEST)
        vec = jnp.concatenate([top, bottom], axis=1)
        stage[pl.ds(base, _CHUNK)] = jnp.broadcast_to(
            vec[:, None, :], (_CHUNK, b, 2 * d))
        pltpu.make_async_copy(
            stage.at[pl.ds(base, _CHUNK)],
            out_ref.at[pl.ds(base, _CHUNK)],
            sems.at[c]).start()
    for c in range(_NCHUNK):
        pltpu.make_async_copy(
            stage.at[pl.ds(c * _CHUNK, _CHUNK)],
            out_ref.at[pl.ds(c * _CHUNK, _CHUNK)],
            sems.at[c]).wait()


def kernel(x, row_embed, col_embed):
    b = x.shape[0]
    h, w = x.shape[2], x.shape[3]
    n, d = row_embed.shape
    hw = h * w
    body = functools.partial(_pos_kernel, h=h, w=w, d=d)
    out = pl.pallas_call(
        body,
        in_specs=[
            pl.BlockSpec((n, d), lambda: (0, 0)),
            pl.BlockSpec((n, d), lambda: (0, 0)),
        ],
        out_specs=pl.BlockSpec(memory_space=pltpu.MemorySpace.HBM),
        out_shape=jax.ShapeDtypeStruct((hw, b, 2 * d), jnp.float32),
        scratch_shapes=[
            pltpu.VMEM((hw, b, 2 * d), jnp.float32),
            pltpu.SemaphoreType.DMA((_NCHUNK,)),
        ],
    )(row_embed, col_embed)
    return jnp.transpose(out.reshape(h, w, b, 2 * d), (2, 3, 0, 1))
